# baseline (speedup 1.0000x reference)
import functools

import jax
import jax.numpy as jnp
from jax import lax
from jax.experimental import pallas as pl
from jax.experimental.pallas import tpu as pltpu
from jax.experimental.pallas import tpu_sc as plsc

NUM_CORES = 2
NUM_SUBCORES = 16
NUM_WORKERS = NUM_CORES * NUM_SUBCORES

CHUNK = 16
NBUF = 6
LOOKAHEAD = 3


@functools.partial(jax.jit, static_argnames=())
def _gather_rows(flat_idx, table):
    n = flat_idx.shape[0]
    d = table.shape[1]
    n_per_w = n // NUM_WORKERS
    n_chunks = n_per_w // CHUNK

    mesh = plsc.VectorSubcoreMesh(core_axis_name="c", subcore_axis_name="s")
    G = LOOKAHEAD

    @functools.partial(
        pl.kernel,
        mesh=mesh,
        out_type=jax.ShapeDtypeStruct((n, d), jnp.float32),
        scratch_types=[
            pltpu.VMEM((n_per_w,), jnp.int32),
            *[pltpu.VMEM((CHUNK, d), jnp.float32) for _ in range(NBUF)],
            *[pltpu.SemaphoreType.DMA for _ in range(2 * NBUF)],
        ],
    )
    def k(idx_hbm, table_hbm, out_hbm, idx_v, *bufs_and_sems):
        bufs = bufs_and_sems[:NBUF]
        gsems = bufs_and_sems[NBUF : 2 * NBUF]
        wsems = bufs_and_sems[2 * NBUF :]

        wid = lax.axis_index("s") * NUM_CORES + lax.axis_index("c")
        base = wid * n_per_w
        pltpu.sync_copy(idx_hbm.at[pl.ds(base, n_per_w)], idx_v)

        def gather(i, b):
            pltpu.async_copy(
                table_hbm.at[idx_v.at[pl.ds(i * CHUNK, CHUNK)]], bufs[b], gsems[b]
            )

        def wait_gather(i, b):
            pltpu.make_async_copy(
                table_hbm.at[idx_v.at[pl.ds(i * CHUNK, CHUNK)]], bufs[b], gsems[b]
            ).wait()

        def write(i, b):
            pltpu.async_copy(
                bufs[b], out_hbm.at[pl.ds(base + i * CHUNK, CHUNK)], wsems[b]
            )

        def wait_write(i, b):
            pltpu.make_async_copy(
                bufs[b], out_hbm.at[pl.ds(base + i * CHUNK, CHUNK)], wsems[b]
            ).wait()

        for j in range(G):
            gather(j, j)

        def body(i, carry):
            @pl.when(i + G - NBUF >= 0)
            def _():
                for b in range(NBUF):

                    @pl.when(lax.rem(i + G - NBUF, NBUF) == b)
                    def _():
                        wait_write(i + G - NBUF, b)

            @pl.when(i + G < n_chunks)
            def _():
                for b in range(NBUF):

                    @pl.when(lax.rem(i + G, NBUF) == b)
                    def _():
                        gather(i + G, b)

            for b in range(NBUF):

                @pl.when(lax.rem(i, NBUF) == b)
                def _():
                    wait_gather(i, b)
                    write(i, b)

            return carry

        lax.fori_loop(0, n_chunks, body, 0)
        for j in range(max(0, n_chunks + G - NBUF), n_chunks):
            wait_write(j, j % NBUF)

    return k(flat_idx, table)


def kernel(input_ids, word_embeddings):
    b, s = input_ids.shape
    d = word_embeddings.shape[1]
    flat_idx = input_ids.reshape(b * s).astype(jnp.int32)
    out = _gather_rows(flat_idx, word_embeddings)
    return out.reshape(b, s, d)


# speedup vs baseline: 1.0180x; 1.0180x over previous
import functools

import jax
import jax.numpy as jnp
from jax import lax
from jax.experimental import pallas as pl
from jax.experimental.pallas import tpu as pltpu
from jax.experimental.pallas import tpu_sc as plsc

NUM_CORES = 2
NUM_SUBCORES = 16
NUM_WORKERS = NUM_CORES * NUM_SUBCORES

CHUNK = 16
NBUF = 3
LOOKAHEAD = 2


@functools.partial(jax.jit, static_argnames=())
def _gather_rows(flat_idx, table):
    n = flat_idx.shape[0]
    d = table.shape[1]
    n_per_w = n // NUM_WORKERS
    n_chunks = n_per_w // CHUNK

    mesh = plsc.VectorSubcoreMesh(core_axis_name="c", subcore_axis_name="s")
    G = LOOKAHEAD

    @functools.partial(
        pl.kernel,
        mesh=mesh,
        out_type=jax.ShapeDtypeStruct((n, d), jnp.float32),
        scratch_types=[
            pltpu.VMEM((n_per_w,), jnp.int32),
            *[pltpu.VMEM((CHUNK, d), jnp.float32) for _ in range(NBUF)],
            pltpu.VMEM_SHARED((NUM_SUBCORES, NBUF, CHUNK, d), jnp.float32),
            *[pltpu.SemaphoreType.DMA for _ in range(3 * NBUF)],
        ],
    )
    def k(idx_hbm, table_hbm, out_hbm, idx_v, *rest):
        tbufs = rest[:NBUF]
        shared = rest[NBUF]
        sems = rest[NBUF + 1 :]
        gsems = sems[:NBUF]
        tsems = sems[NBUF : 2 * NBUF]
        wsems = sems[2 * NBUF :]

        sid = lax.axis_index("s")
        wid = sid * NUM_CORES + lax.axis_index("c")
        base = wid * n_per_w
        pltpu.sync_copy(idx_hbm.at[pl.ds(base, n_per_w)], idx_v)

        def gather(i, b):
            pltpu.async_copy(
                table_hbm.at[idx_v.at[pl.ds(i * CHUNK, CHUNK)]], tbufs[b], gsems[b]
            )

        def wait_gather(i, b):
            pltpu.make_async_copy(
                table_hbm.at[idx_v.at[pl.ds(i * CHUNK, CHUNK)]], tbufs[b], gsems[b]
            ).wait()

        def t2s(i, b):
            pltpu.async_copy(tbufs[b], shared.at[sid, b], tsems[b])

        def wait_t2s(i, b):
            pltpu.make_async_copy(tbufs[b], shared.at[sid, b], tsems[b]).wait()

        def write(i, b):
            pltpu.async_copy(
                shared.at[sid, b], out_hbm.at[pl.ds(base + i * CHUNK, CHUNK)], wsems[b]
            )

        def wait_write(i, b):
            pltpu.make_async_copy(
                shared.at[sid, b], out_hbm.at[pl.ds(base + i * CHUNK, CHUNK)], wsems[b]
            ).wait()

        def sel(i, fn):
            for b in range(NBUF):

                @pl.when(lax.rem(i, NBUF) == b)
                def _():
                    fn(i, b)

        for j in range(min(2, n_chunks)):
            gather(j, j % NBUF)

        def body(i, carry):
            @pl.when(i - NBUF >= 0)
            def _():
                sel(i - NBUF, wait_write)

            @pl.when(i - 1 >= 0)
            def _():
                sel(i - 1, wait_t2s)
                sel(i - 1, write)

            @pl.when(i + 2 < n_chunks)
            def _():
                sel(i + 2, gather)

            sel(i, wait_gather)
            sel(i, t2s)

            return carry

        lax.fori_loop(0, n_chunks, body, 0)
        sel(n_chunks - 1, wait_t2s)
        sel(n_chunks - 1, write)
        for j in range(max(0, n_chunks - NBUF), n_chunks):
            sel(j, wait_write)

    return k(flat_idx, table)


def kernel(input_ids, word_embeddings):
    b, s = input_ids.shape
    d = word_embeddings.shape[1]
    flat_idx = input_ids.reshape(b * s).astype(jnp.int32)
    out = _gather_rows(flat_idx, word_embeddings)
    return out.reshape(b, s, d)
